# baseline (speedup 1.0000x reference)
import functools

import jax
import jax.numpy as jnp
from jax import lax
from jax.experimental import pallas as pl
from jax.experimental.pallas import tpu as pltpu
from jax.experimental.pallas import tpu_sc as plsc

_ROWS = (0, 2, 1)

_info = plsc.get_sparse_core_info()
_NC = _info.num_cores

_mesh = plsc.VectorSubcoreMesh(core_axis_name="c", subcore_axis_name="s")


@functools.partial(
    pl.kernel,
    out_type=jax.ShapeDtypeStruct((len(_ROWS), 128), jnp.float32),
    mesh=_mesh,
)
def _gather_rows(x_hbm, out_hbm):
    wid = lax.axis_index("s") * _NC + lax.axis_index("c")
    for j, src in enumerate(_ROWS):
        @pl.when(wid == j)
        def _copy(j=j, src=src):
            pltpu.sync_copy(x_hbm.at[pl.ds(src, 1)], out_hbm.at[pl.ds(j, 1)])


def kernel(x):
    return _gather_rows(x)


# speedup vs baseline: 1.0005x; 1.0005x over previous
import functools

import jax
import jax.numpy as jnp
from jax import lax
from jax.experimental import pallas as pl
from jax.experimental.pallas import tpu as pltpu
from jax.experimental.pallas import tpu_sc as plsc

_ROWS = (0, 2, 1)

_mesh = plsc.ScalarSubcoreMesh(axis_name="c")


@functools.partial(
    pl.kernel,
    out_type=jax.ShapeDtypeStruct((len(_ROWS), 128), jnp.float32),
    mesh=_mesh,
)
def _gather_rows(x_hbm, out_hbm):
    cid = lax.axis_index("c")

    @pl.when(cid == 0)
    def _():
        for j, src in enumerate(_ROWS):
            pltpu.sync_copy(x_hbm.at[pl.ds(src, 1)], out_hbm.at[pl.ds(j, 1)])


def kernel(x):
    return _gather_rows(x)


# speedup vs baseline: 1.1879x; 1.1872x over previous
import functools

import jax
import jax.numpy as jnp
from jax.experimental import pallas as pl
from jax.experimental.pallas import tpu as pltpu
from jax.experimental.pallas import tpu_sc as plsc

_ROWS = (0, 2, 1)

_mesh = plsc.ScalarSubcoreMesh(axis_name="c", num_cores=1)


@functools.partial(
    pl.kernel,
    out_type=jax.ShapeDtypeStruct((len(_ROWS), 128), jnp.float32),
    mesh=_mesh,
    scratch_types=[pltpu.SemaphoreType.DMA],
)
def _gather_rows(x_hbm, out_hbm, sem):
    copies = [
        pltpu.make_async_copy(
            x_hbm.at[pl.ds(src, 1)], out_hbm.at[pl.ds(j, 1)], sem
        )
        for j, src in enumerate(_ROWS)
    ]
    for cp in copies:
        cp.start()
    for cp in copies:
        cp.wait()


def kernel(x):
    return _gather_rows(x)


# speedup vs baseline: 1.1940x; 1.0051x over previous
import functools

import jax
import jax.numpy as jnp
from jax.experimental import pallas as pl
from jax.experimental.pallas import tpu as pltpu
from jax.experimental.pallas import tpu_sc as plsc

_ROWS = (0, 2, 1)

_mesh = plsc.ScalarSubcoreMesh(axis_name="c", num_cores=1)


@functools.partial(
    pl.kernel,
    out_type=jax.ShapeDtypeStruct((len(_ROWS), 128), jnp.float32),
    mesh=_mesh,
    scratch_types=[pltpu.SemaphoreType.DMA],
)
def _gather_rows(x_hbm, out_hbm, sem):
    copies = [
        pltpu.make_async_copy(
            x_hbm.at[pl.ds(src, 1)], out_hbm.at[pl.ds(j, 1)], sem
        )
        for j, src in enumerate(_ROWS)
    ]
    for cp in copies:
        cp.start()
    pltpu.make_async_copy(x_hbm.at[pl.ds(0, len(_ROWS))], out_hbm, sem).wait()


def kernel(x):
    return _gather_rows(x)
